# baseline (speedup 1.0000x reference)
import functools

import jax
import jax.numpy as jnp
from jax.experimental import pallas as pl
from jax.experimental.pallas import tpu as pltpu

_NB = 4


@functools.cache
def _build(B, T, N, C, K):
    def body(idx_ref, patch_hbm, audio_hbm, outp_hbm, outa_hbm,
             bufs, abuf, insems, outsems, asem_i, asem_o):
        del bufs, abuf, insems, outsems, asem_o
        nslices = B * K
        hs = []
        for s in range(nslices):
            b, k = divmod(s, K)
            t = idx_ref[b, k]
            h = pltpu.make_async_copy(patch_hbm.at[b, t], outp_hbm.at[b, k],
                                      outsems.at[s % _NB] if False else asem_i)
            h.start()
            hs.append(h)
            ha = pltpu.make_async_copy(audio_hbm.at[b, t], outa_hbm.at[s], asem_i)
            ha.start()
            hs.append(ha)
        for h in hs:
            h.wait()

    grid_spec = pltpu.PrefetchScalarGridSpec(
        num_scalar_prefetch=1,
        grid=(1,),
        in_specs=[
            pl.BlockSpec(memory_space=pl.ANY),
            pl.BlockSpec(memory_space=pl.ANY),
        ],
        out_specs=[
            pl.BlockSpec(memory_space=pl.ANY),
            pl.BlockSpec(memory_space=pl.ANY),
        ],
        scratch_shapes=[
            pltpu.VMEM((_NB, N, C), jnp.float32),
            pltpu.VMEM((B * K, C), jnp.float32),
            pltpu.SemaphoreType.DMA((_NB,)),
            pltpu.SemaphoreType.DMA((_NB,)),
            pltpu.SemaphoreType.DMA,
            pltpu.SemaphoreType.DMA,
        ],
    )
    return pl.pallas_call(
        body,
        grid_spec=grid_spec,
        out_shape=[
            jax.ShapeDtypeStruct((B, K, N, C), jnp.float32),
            jax.ShapeDtypeStruct((B * K, C), jnp.float32),
        ],
    )


def kernel(top_k_index_sort, patch_feat, audio_feat):
    B, T, N, C = patch_feat.shape
    K = top_k_index_sort.shape[-1]
    idx = top_k_index_sort.reshape(B, K).astype(jnp.int32)
    out_p, out_a = _build(B, T, N, C, K)(idx, patch_feat, audio_feat)
    return out_p, out_a.reshape(B, K, C)


# speedup vs baseline: 26.7115x; 26.7115x over previous
import functools

import jax
import jax.numpy as jnp
from jax.experimental import pallas as pl
from jax.experimental.pallas import tpu as pltpu

_NB = 8


@functools.cache
def _build(B, T, N, C, K):
    def body(idx_ref, patch_hbm, audio_hbm, outp_hbm, outa_hbm,
             bufs, abuf, insems, outsems, asem_i, asem_o):
        nslices = B * K
        in_h = [None] * nslices
        out_h = [None] * nslices

        def start_in(s):
            b, k = divmod(s, K)
            t = idx_ref[b, k]
            r = s % _NB
            return pltpu.make_async_copy(
                patch_hbm.at[b, t], bufs.at[r], insems.at[r])

        def start_out(s):
            b, k = divmod(s, K)
            r = s % _NB
            return pltpu.make_async_copy(
                bufs.at[r], outp_hbm.at[b, k], outsems.at[r])

        ah_in = [None] * nslices
        for s in range(nslices):
            b, k = divmod(s, K)
            t = idx_ref[b, k]
            h = pltpu.make_async_copy(audio_hbm.at[b, t], abuf.at[s], asem_i)
            h.start()
            ah_in[s] = h

        for s in range(nslices):
            r = s % _NB
            if s >= _NB:
                out_h[s - _NB].wait()
            h = start_in(s)
            h.start()
            in_h[s] = h
            if s >= 1:
                in_h[s - 1].wait()
                oh = start_out(s - 1)
                oh.start()
                out_h[s - 1] = oh
        in_h[nslices - 1].wait()
        oh = start_out(nslices - 1)
        oh.start()
        out_h[nslices - 1] = oh
        for s in range(max(0, nslices - _NB), nslices):
            out_h[s].wait()

        for s in range(nslices):
            ah_in[s].wait()
        ao = pltpu.make_async_copy(abuf, outa_hbm, asem_o)
        ao.start()
        ao.wait()

    grid_spec = pltpu.PrefetchScalarGridSpec(
        num_scalar_prefetch=1,
        grid=(1,),
        in_specs=[
            pl.BlockSpec(memory_space=pl.ANY),
            pl.BlockSpec(memory_space=pl.ANY),
        ],
        out_specs=[
            pl.BlockSpec(memory_space=pl.ANY),
            pl.BlockSpec(memory_space=pl.ANY),
        ],
        scratch_shapes=[
            pltpu.VMEM((_NB, N, C), jnp.float32),
            pltpu.VMEM((B * K, C), jnp.float32),
            pltpu.SemaphoreType.DMA((_NB,)),
            pltpu.SemaphoreType.DMA((_NB,)),
            pltpu.SemaphoreType.DMA,
            pltpu.SemaphoreType.DMA,
        ],
    )
    return pl.pallas_call(
        body,
        grid_spec=grid_spec,
        out_shape=[
            jax.ShapeDtypeStruct((B, K, N, C), jnp.float32),
            jax.ShapeDtypeStruct((B * K, C), jnp.float32),
        ],
    )


def kernel(top_k_index_sort, patch_feat, audio_feat):
    B, T, N, C = patch_feat.shape
    K = top_k_index_sort.shape[-1]
    idx = top_k_index_sort.reshape(B, K).astype(jnp.int32)
    out_p, out_a = _build(B, T, N, C, K)(idx, patch_feat, audio_feat)
    return out_p, out_a.reshape(B, K, C)
